# baseline (speedup 1.0000x reference)
import functools

import jax
import jax.numpy as jnp
from jax.experimental import pallas as pl

MASK_PROP = 0.75


def _mask_encode_kernel(num_mask, p_ref, rl_ref, rc_ref, m_ref, e_ref, i_ref):
    n = p_ref.shape[1]
    num_unmask = n - num_mask
    chunk = rc_ref.shape[1]
    cols = rc_ref.shape[2]

    kk = jax.lax.broadcasted_iota(jnp.int32, (num_unmask, n), 0) + num_mask
    col = jax.lax.broadcasted_iota(jnp.int32, (chunk, n), 1)
    for bb in range(p_ref.shape[0]):
        v = rl_ref[bb, 0, :][None, :]
        rank = jnp.zeros((1, n), jnp.float32)
        for t in range(cols):
            vj = rc_ref[bb, :, t : t + 1]
            rank = rank + jnp.sum(
                (vj < v).astype(jnp.float32), axis=0, keepdims=True
            )
        ranki = rank.astype(jnp.int32)

        onehot = (ranki == kk).astype(jnp.float32)
        e_ref[bb, :num_unmask, :] = jnp.dot(
            onehot, p_ref[bb], preferred_element_type=jnp.float32
        )
        e_ref[bb, num_unmask:, :] = jnp.broadcast_to(
            m_ref[0, :], (num_mask, e_ref.shape[2])
        )

        for c in range(0, num_mask, chunk):
            mk = jax.lax.broadcasted_iota(jnp.int32, (chunk, n), 0) + c
            sel = ranki == mk
            i_ref[bb, 0, c : c + chunk] = jnp.sum(jnp.where(sel, col, 0), axis=1)


def kernel(patches, mask_token):
    b, n, e = patches.shape
    num_mask = -(-3 * n // 4)

    rkey = jax.random.key(42)
    rand_vals = jax.random.uniform(rkey, (b, n), dtype=jnp.float32)
    rand_lane = rand_vals.reshape(b, 1, n)
    rand_cols = rand_vals.reshape(b, n // 12, 12)

    bb = 8
    enc, idx3 = pl.pallas_call(
        functools.partial(_mask_encode_kernel, num_mask),
        grid=(b // bb,),
        in_specs=[
            pl.BlockSpec((bb, n, e), lambda i: (i, 0, 0)),
            pl.BlockSpec((bb, 1, n), lambda i: (i, 0, 0)),
            pl.BlockSpec((bb, n // 12, 12), lambda i: (i, 0, 0)),
            pl.BlockSpec((1, e), lambda i: (0, 0)),
        ],
        out_specs=[
            pl.BlockSpec((bb, n, e), lambda i: (i, 0, 0)),
            pl.BlockSpec((bb, 1, num_mask), lambda i: (i, 0, 0)),
        ],
        out_shape=[
            jax.ShapeDtypeStruct((b, n, e), jnp.float32),
            jax.ShapeDtypeStruct((b, 1, num_mask), jnp.int32),
        ],
    )(patches, rand_lane, rand_cols, mask_token)
    return enc, idx3.reshape(b, num_mask)
